# Initial kernel scaffold; baseline (speedup 1.0000x reference)
import functools

import jax
import jax.numpy as jnp
from jax import lax
from jax.experimental import pallas as pl
from jax.experimental.pallas import tpu as pltpu
from jax.experimental.pallas import tpu_sc as plsc

N = 10000
E = 160000
D = 256
K = 3
NC = 2
NS = 16
L = 16
DH = D // NC
FV = DH // L
EPT = E // NS
C = 80
NCH = EPT // C
RPT = N // NS
NRC = 25
NQ = RPT // NRC


def _rsqrt_newton(d):
    hd = d * 0.5
    i = plsc.bitcast(d, jnp.int32)
    i = 0x5F3759DF - lax.shift_right_arithmetic(i, 1)
    y = plsc.bitcast(i, jnp.float32)
    for _ in range(4):
        y = y * (1.5 - hd * y * y)
    return jnp.where(d > 0.0, y, 0.0)


def _lightgcn_body(x_hbm, ei_hbm, out_hbm, hs_hbm,
                   acc, degw,
                   outsum, gbuf, rowb, colb, sw, nbuf, zbuf, zdeg, onesb,
                   sem_i, sem_g, sem_s):
    c = lax.axis_index("c")
    s = lax.axis_index("s")
    r0t = s * RPT
    e0t = s * EPT
    c0 = c * DH

    zero16 = jnp.zeros((L,), jnp.float32)
    one16 = jnp.ones((L,), jnp.float32)

    @pl.loop(0, NRC)
    def _(r):
        for f in range(FV):
            zbuf[r, pl.ds(f * L, L)] = zero16

    @pl.loop(0, 125)
    def _(r):
        zdeg[r] = zero16

    @pl.loop(0, C)
    def _(r):
        onesb[r] = one16

    for q in range(RPT // 125):
        pltpu.sync_copy(zdeg, degw.at[pl.ds(r0t + q * 125, 125)])

    @pl.loop(0, NQ)
    def _(q):
        pltpu.sync_copy(zbuf, acc.at[pl.ds(r0t + q * NRC, NRC)])

    plsc.subcore_barrier()

    pltpu.sync_copy(ei_hbm.at[0, pl.ds(e0t, C)], rowb.at[0])

    @pl.loop(0, NCH)
    def _(j):
        b = lax.rem(j, 2)
        nb = lax.rem(j + 1, 2)

        @pl.when(j < NCH - 1)
        def _():
            e0 = pl.multiple_of(e0t + (j + 1) * C, 8)
            pltpu.async_copy(ei_hbm.at[0, pl.ds(e0, C)], rowb.at[nb], sem_i)

        pltpu.sync_copy(onesb, degw.at[rowb.at[b]], add=True)

        @pl.when(j < NCH - 1)
        def _():
            pltpu.make_async_copy(
                ei_hbm.at[0, pl.ds(e0t, C)], rowb.at[nb], sem_i).wait()

    plsc.subcore_barrier()

    pltpu.sync_copy(degw.at[pl.ds(r0t, RPT)], sw)

    @pl.loop(0, RPT)
    def _(r):
        sw[r] = _rsqrt_newton(sw[r])

    @pl.loop(0, NQ)
    def _(q):
        r0g = r0t + q * NRC
        r0l = q * NRC
        pltpu.sync_copy(x_hbm.at[pl.ds(r0g, NRC), pl.ds(c0, DH)], nbuf)

        @pl.loop(0, NRC)
        def _(r):
            sv = sw[r0l + r]
            for f in range(FV):
                df = pl.ds(f * L, L)
                v = nbuf[r, df]
                outsum[r0l + r, df] = v
                nbuf[r, df] = v * sv

        pltpu.sync_copy(nbuf, hs_hbm.at[c].at[pl.ds(r0g, NRC)])

    plsc.subcore_barrier()

    for k in range(1, K + 1):
        pltpu.sync_copy(ei_hbm.at[0, pl.ds(e0t, C)], rowb.at[0])
        pltpu.sync_copy(ei_hbm.at[1, pl.ds(e0t, C)], colb.at[0])
        pltpu.async_copy(hs_hbm.at[c].at[colb.at[0]], gbuf.at[0], sem_g)

        @pl.loop(0, NCH)
        def _(j):
            b = lax.rem(j, 2)
            nb = lax.rem(j + 1, 2)

            @pl.when(j > 0)
            def _():
                pltpu.make_async_copy(
                    gbuf.at[nb], acc.at[rowb.at[nb]], sem_s).wait()

            @pl.when(j < NCH - 1)
            def _():
                e0 = pl.multiple_of(e0t + (j + 1) * C, 8)
                pltpu.async_copy(ei_hbm.at[0, pl.ds(e0, C)], rowb.at[nb],
                                 sem_i)
                pltpu.async_copy(ei_hbm.at[1, pl.ds(e0, C)], colb.at[nb],
                                 sem_i)

            pltpu.make_async_copy(
                hs_hbm.at[c].at[colb.at[b]], gbuf.at[b], sem_g).wait()
            pltpu.async_copy(gbuf.at[b], acc.at[rowb.at[b]], sem_s, add=True)

            @pl.when(j < NCH - 1)
            def _():
                pltpu.make_async_copy(
                    ei_hbm.at[0, pl.ds(e0t, C)], rowb.at[nb], sem_i).wait()
                pltpu.make_async_copy(
                    ei_hbm.at[1, pl.ds(e0t, C)], colb.at[nb], sem_i).wait()
                pltpu.async_copy(hs_hbm.at[c].at[colb.at[nb]], gbuf.at[nb],
                                 sem_g)

        lastb = (NCH - 1) % 2
        pltpu.make_async_copy(
            gbuf.at[lastb], acc.at[rowb.at[lastb]], sem_s).wait()

        plsc.subcore_barrier()

        @pl.loop(0, NQ)
        def _(q):
            r0g = r0t + q * NRC
            r0l = q * NRC
            pltpu.sync_copy(acc.at[pl.ds(r0g, NRC)], nbuf)
            if k < K:
                pltpu.sync_copy(zbuf, acc.at[pl.ds(r0g, NRC)])

            @pl.loop(0, NRC)
            def _(r):
                sv = sw[r0l + r]
                s2v = sv * sv
                for f in range(FV):
                    df = pl.ds(f * L, L)
                    a = nbuf[r, df]
                    h = a * sv
                    if k < K:
                        outsum[r0l + r, df] = outsum[r0l + r, df] + h
                        nbuf[r, df] = a * s2v
                    else:
                        nbuf[r, df] = (outsum[r0l + r, df] + h) * 0.25

            if k < K:
                pltpu.sync_copy(nbuf, hs_hbm.at[c].at[pl.ds(r0g, NRC)])
            else:
                pltpu.sync_copy(
                    nbuf, out_hbm.at[pl.ds(r0g, NRC), pl.ds(c0, DH)])

        if k < K:
            plsc.subcore_barrier()


@functools.partial(
    pl.kernel,
    out_type=(
        jax.ShapeDtypeStruct((N, D), jnp.float32),
        jax.ShapeDtypeStruct((NC, N, DH), jnp.float32),
    ),
    mesh=plsc.VectorSubcoreMesh(core_axis_name="c", subcore_axis_name="s"),
    scratch_types=[
        pltpu.VMEM_SHARED((N, DH), jnp.float32),
        pltpu.VMEM_SHARED((N, L), jnp.float32),
        pltpu.VMEM((RPT, DH), jnp.float32),
        pltpu.VMEM((2, C, DH), jnp.float32),
        pltpu.VMEM((2, C), jnp.int32),
        pltpu.VMEM((2, C), jnp.int32),
        pltpu.VMEM((RPT, L), jnp.float32),
        pltpu.VMEM((NRC, DH), jnp.float32),
        pltpu.VMEM((NRC, DH), jnp.float32),
        pltpu.VMEM((125, L), jnp.float32),
        pltpu.VMEM((C, L), jnp.float32),
        pltpu.SemaphoreType.DMA,
        pltpu.SemaphoreType.DMA,
        pltpu.SemaphoreType.DMA,
    ],
)
def _lightgcn_sc(*refs):
    _lightgcn_body(*refs)


def kernel(x, edge_index):
    out, _ = _lightgcn_sc(x, edge_index)
    return out


# speedup vs baseline: 5.0037x; 5.0037x over previous
import functools

import jax
import jax.numpy as jnp
from jax import lax
from jax.experimental import pallas as pl
from jax.experimental.pallas import tpu as pltpu
from jax.experimental.pallas import tpu_sc as plsc

N = 10000
NP = 10240
E = 160000
D = 256
K = 3
NC = 2
NS = 16
L = 16
DQ = 64
FV = DQ // L
EPT = E // NS
C = 80
NCH = EPT // C
RPT = NP // NS
NRC = 40
NQ = RPT // NRC
ZR = 128


def _deg_inv_sqrt(d):
    dm = jnp.maximum(d, 1.0)
    s = dm
    for _ in range(16):
        s = 0.5 * (s + dm / s)
    return jnp.where(d > 0.0, 1.0 / s, 0.0)


def _lightgcn_body(x_hbm, row_hbm, col_hbm, out_hbm, hs_hbm, osum_hbm,
                   acc,
                   gbuf, rowb, colb, sw, nbuf, zbuf, osbuf, onesb,
                   sem_i, sem_g, sem_s):
    c = lax.axis_index("c")
    s = lax.axis_index("s")
    r0t = s * RPT
    e0t = s * EPT

    zero16 = jnp.zeros((L,), jnp.float32)
    one16 = jnp.ones((L,), jnp.float32)

    @pl.loop(0, NRC)
    def _(r):
        for f in range(FV):
            zbuf[r, pl.ds(f * L, L)] = zero16

    @pl.loop(0, C)
    def _(r):
        for f in range(FV):
            onesb[r, pl.ds(f * L, L)] = one16

    @pl.loop(0, NQ)
    def _(q):
        pltpu.sync_copy(zbuf, acc.at[pl.ds(r0t + q * NRC, NRC)])

    plsc.subcore_barrier()

    pltpu.sync_copy(row_hbm.at[pl.ds(e0t, C)], rowb.at[0])

    @pl.loop(0, NCH)
    def _(j):
        b = lax.rem(j, 2)
        nb = lax.rem(j + 1, 2)

        @pl.when(j < NCH - 1)
        def _():
            e0 = pl.multiple_of(e0t + (j + 1) * C, 8)
            pltpu.async_copy(row_hbm.at[pl.ds(e0, C)], rowb.at[nb], sem_i)

        pltpu.sync_copy(onesb, acc.at[rowb.at[b]], add=True)

        @pl.when(j < NCH - 1)
        def _():
            pltpu.make_async_copy(
                row_hbm.at[pl.ds(e0t, C)], rowb.at[nb], sem_i).wait()

    plsc.subcore_barrier()

    @pl.loop(0, NQ)
    def _(q):
        r0g = r0t + q * NRC
        r0l = q * NRC
        pltpu.sync_copy(acc.at[pl.ds(r0g, NRC)], nbuf)
        pltpu.sync_copy(zbuf, acc.at[pl.ds(r0g, NRC)])

        @pl.loop(0, NRC)
        def _(r):
            sw[r0l + r] = nbuf[r, pl.ds(0, L)]

    @pl.loop(0, RPT)
    def _(r):
        sw[r] = _deg_inv_sqrt(sw[r])

    for half in range(2):
        qidx = c * 2 + half

        @pl.loop(0, NQ)
        def _(q):
            r0g = r0t + q * NRC
            r0l = q * NRC
            pltpu.sync_copy(x_hbm.at[qidx].at[pl.ds(r0g, NRC)], nbuf)
            pltpu.sync_copy(nbuf, osum_hbm.at[qidx].at[pl.ds(r0g, NRC)])

            @pl.loop(0, NRC)
            def _(r):
                sv = sw[r0l + r]
                for f in range(FV):
                    df = pl.ds(f * L, L)
                    nbuf[r, df] = nbuf[r, df] * sv

            pltpu.sync_copy(nbuf, hs_hbm.at[qidx].at[pl.ds(r0g, NRC)])

    plsc.subcore_barrier()

    for k in range(1, K + 1):
        for half in range(2):
            qidx = c * 2 + half
            last_pass = (k == K) and (half == 1)

            pltpu.sync_copy(row_hbm.at[pl.ds(e0t, C)], rowb.at[0])
            pltpu.sync_copy(col_hbm.at[pl.ds(e0t, C)], colb.at[0])
            pltpu.async_copy(hs_hbm.at[qidx].at[colb.at[0]], gbuf.at[0],
                             sem_g)

            @pl.loop(0, NCH)
            def _(j):
                b = lax.rem(j, 2)
                nb = lax.rem(j + 1, 2)

                @pl.when(j > 0)
                def _():
                    pltpu.make_async_copy(
                        gbuf.at[nb], acc.at[rowb.at[nb]], sem_s).wait()

                @pl.when(j < NCH - 1)
                def _():
                    e0 = pl.multiple_of(e0t + (j + 1) * C, 8)
                    pltpu.async_copy(row_hbm.at[pl.ds(e0, C)], rowb.at[nb],
                                     sem_i)
                    pltpu.async_copy(col_hbm.at[pl.ds(e0, C)], colb.at[nb],
                                     sem_i)

                pltpu.make_async_copy(
                    hs_hbm.at[qidx].at[colb.at[b]], gbuf.at[b], sem_g).wait()
                pltpu.async_copy(gbuf.at[b], acc.at[rowb.at[b]], sem_s,
                                 add=True)

                @pl.when(j < NCH - 1)
                def _():
                    pltpu.make_async_copy(
                        row_hbm.at[pl.ds(e0t, C)], rowb.at[nb], sem_i).wait()
                    pltpu.make_async_copy(
                        col_hbm.at[pl.ds(e0t, C)], colb.at[nb], sem_i).wait()
                    pltpu.async_copy(hs_hbm.at[qidx].at[colb.at[nb]],
                                     gbuf.at[nb], sem_g)

            lastb = (NCH - 1) % 2
            pltpu.make_async_copy(
                gbuf.at[lastb], acc.at[rowb.at[lastb]], sem_s).wait()

            plsc.subcore_barrier()

            @pl.loop(0, NQ)
            def _(q):
                r0g = r0t + q * NRC
                r0l = q * NRC
                pltpu.sync_copy(acc.at[pl.ds(r0g, NRC)], nbuf)
                if not last_pass:
                    pltpu.sync_copy(zbuf, acc.at[pl.ds(r0g, NRC)])
                pltpu.sync_copy(osum_hbm.at[qidx].at[pl.ds(r0g, NRC)], osbuf)

                @pl.loop(0, NRC)
                def _(r):
                    sv = sw[r0l + r]
                    s2v = sv * sv
                    for f in range(FV):
                        df = pl.ds(f * L, L)
                        a = nbuf[r, df]
                        h = a * sv
                        if k < K:
                            osbuf[r, df] = osbuf[r, df] + h
                            nbuf[r, df] = a * s2v
                        else:
                            nbuf[r, df] = (osbuf[r, df] + h) * 0.25

                if k < K:
                    pltpu.sync_copy(osbuf,
                                    osum_hbm.at[qidx].at[pl.ds(r0g, NRC)])
                    pltpu.sync_copy(nbuf, hs_hbm.at[qidx].at[pl.ds(r0g, NRC)])
                else:
                    pltpu.sync_copy(nbuf, out_hbm.at[qidx].at[pl.ds(r0g,
                                                                    NRC)])

            if not last_pass:
                plsc.subcore_barrier()


@functools.partial(
    pl.kernel,
    out_type=(
        jax.ShapeDtypeStruct((4, NP, DQ), jnp.float32),
        jax.ShapeDtypeStruct((4, NP, DQ), jnp.float32),
        jax.ShapeDtypeStruct((4, NP, DQ), jnp.float32),
    ),
    mesh=plsc.VectorSubcoreMesh(core_axis_name="c", subcore_axis_name="s"),
    compiler_params=pltpu.CompilerParams(use_tc_tiling_on_sc=False),
    scratch_types=[
        pltpu.VMEM_SHARED((NP, DQ), jnp.float32),
        pltpu.VMEM((2, C, DQ), jnp.float32),
        pltpu.VMEM((2, C), jnp.int32),
        pltpu.VMEM((2, C), jnp.int32),
        pltpu.VMEM((RPT, L), jnp.float32),
        pltpu.VMEM((NRC, DQ), jnp.float32),
        pltpu.VMEM((NRC, DQ), jnp.float32),
        pltpu.VMEM((NRC, DQ), jnp.float32),
        pltpu.VMEM((C, DQ), jnp.float32),
        pltpu.SemaphoreType.DMA,
        pltpu.SemaphoreType.DMA,
        pltpu.SemaphoreType.DMA,
    ],
)
def _lightgcn_sc(*refs):
    _lightgcn_body(*refs)


def kernel(x, edge_index):
    x_p = jnp.pad(x, ((0, NP - N), (0, 0)))
    x_q = jnp.transpose(x_p.reshape(NP, 4, DQ), (1, 0, 2))
    out_q, _, _ = _lightgcn_sc(x_q, edge_index[0], edge_index[1])
    out_p = jnp.transpose(out_q, (1, 0, 2)).reshape(NP, D)
    return out_p[:N]


# speedup vs baseline: 10.4818x; 2.0948x over previous
import functools

import jax
import jax.numpy as jnp
from jax import lax
from jax.experimental import pallas as pl
from jax.experimental.pallas import tpu as pltpu
from jax.experimental.pallas import tpu_sc as plsc

N = 10000
E = 160000
D = 256
K = 3
NC = 2
NS = 16
L = 16
DQ = 64
FV = DQ // L
EPT = E // NS
C = 80
NCH = EPT // C
NB = 4
RPT = N // NS
NRC = 125
NQ = RPT // NRC


def _deg_inv_sqrt(d):
    dm = jnp.maximum(d, 1.0)
    s = dm
    for _ in range(16):
        s = 0.5 * (s + dm / s)
    return jnp.where(d > 0.0, 1.0 / s, 0.0)


def _lightgcn_body(x_hbm, row_hbm, col_hbm, out_hbm, hs_hbm, osum_hbm,
                   acc, degw,
                   gbuf, rowb, colb, sw, nbuf, osbuf, zbuf, zdeg, onesb,
                   sem_g, sem_s):
    c = lax.axis_index("c")
    s = lax.axis_index("s")
    r0t = s * RPT
    ech0 = s * NCH

    zero16 = jnp.zeros((L,), jnp.float32)
    one16 = jnp.ones((L,), jnp.float32)

    @pl.loop(0, NRC)
    def _(r):
        for f in range(FV):
            zbuf[r, pl.ds(f * L, L)] = zero16

    @pl.loop(0, NRC)
    def _(r):
        zdeg[r] = zero16

    @pl.loop(0, C)
    def _(r):
        onesb[r] = one16

    pltpu.sync_copy(row_hbm.at[pl.ds(ech0, NCH)], rowb)
    pltpu.sync_copy(col_hbm.at[pl.ds(ech0, NCH)], colb)

    @pl.loop(0, NQ)
    def _(q):
        pltpu.sync_copy(zbuf, acc.at[pl.ds(r0t + q * NRC, NRC)])
        pltpu.sync_copy(zdeg, degw.at[pl.ds(r0t + q * NRC, NRC)])

    plsc.subcore_barrier()

    @pl.loop(0, NCH)
    def _(j):
        pltpu.async_copy(onesb, degw.at[rowb.at[j]], sem_s, add=True)

        @pl.when(j >= 3)
        def _():
            pltpu.make_async_copy(
                onesb, degw.at[rowb.at[j - 3]], sem_s).wait()

    for t in range(NCH - 3, NCH):
        pltpu.make_async_copy(onesb, degw.at[rowb.at[t]], sem_s).wait()

    plsc.subcore_barrier()

    pltpu.sync_copy(degw.at[pl.ds(r0t, RPT)], sw)

    @pl.loop(0, RPT)
    def _(r):
        sw[r] = _deg_inv_sqrt(sw[r])

    for half in range(2):
        qidx = c * 2 + half

        @pl.loop(0, NQ)
        def _(q):
            r0g = r0t + q * NRC
            r0l = q * NRC
            pltpu.sync_copy(
                x_hbm.at[pl.ds(r0g, NRC), pl.ds(qidx * DQ, DQ)], nbuf)
            pltpu.sync_copy(nbuf, osum_hbm.at[qidx].at[pl.ds(r0g, NRC)])

            @pl.loop(0, NRC)
            def _(r):
                sv = sw[r0l + r]
                for f in range(FV):
                    df = pl.ds(f * L, L)
                    nbuf[r, df] = nbuf[r, df] * sv

            pltpu.sync_copy(nbuf, hs_hbm.at[qidx].at[pl.ds(r0g, NRC)])

    plsc.subcore_barrier()

    for k in range(1, K + 1):
        for half in range(2):
            qidx = c * 2 + half
            last_pass = (k == K) and (half == 1)

            for t in range(3):
                pltpu.async_copy(hs_hbm.at[qidx].at[colb.at[t]],
                                 gbuf.at[t], sem_g)

            @pl.loop(0, NCH)
            def _(j):
                b = lax.rem(j, NB)
                pb = lax.rem(j + NB - 1, NB)

                pltpu.make_async_copy(
                    hs_hbm.at[qidx].at[colb.at[j]], gbuf.at[b], sem_g).wait()
                pltpu.async_copy(gbuf.at[b], acc.at[rowb.at[j]], sem_s,
                                 add=True)

                @pl.when(j >= 1)
                def _():
                    pltpu.make_async_copy(
                        gbuf.at[pb], acc.at[rowb.at[j - 1]], sem_s).wait()

                @pl.when(j < NCH - 3)
                def _():
                    pltpu.async_copy(hs_hbm.at[qidx].at[colb.at[j + 3]],
                                     gbuf.at[pb], sem_g)

            lastb = (NCH - 1) % NB
            pltpu.make_async_copy(
                gbuf.at[lastb], acc.at[rowb.at[NCH - 1]], sem_s).wait()

            plsc.subcore_barrier()

            @pl.loop(0, NQ)
            def _(q):
                r0g = r0t + q * NRC
                r0l = q * NRC
                pltpu.sync_copy(acc.at[pl.ds(r0g, NRC)], nbuf)
                if not last_pass:
                    pltpu.sync_copy(zbuf, acc.at[pl.ds(r0g, NRC)])
                pltpu.sync_copy(osum_hbm.at[qidx].at[pl.ds(r0g, NRC)], osbuf)

                @pl.loop(0, NRC)
                def _(r):
                    sv = sw[r0l + r]
                    s2v = sv * sv
                    for f in range(FV):
                        df = pl.ds(f * L, L)
                        a = nbuf[r, df]
                        h = a * sv
                        if k < K:
                            osbuf[r, df] = osbuf[r, df] + h
                            nbuf[r, df] = a * s2v
                        else:
                            nbuf[r, df] = (osbuf[r, df] + h) * 0.25

                if k < K:
                    pltpu.sync_copy(osbuf,
                                    osum_hbm.at[qidx].at[pl.ds(r0g, NRC)])
                    pltpu.sync_copy(nbuf, hs_hbm.at[qidx].at[pl.ds(r0g, NRC)])
                else:
                    pltpu.sync_copy(
                        nbuf,
                        out_hbm.at[pl.ds(r0g, NRC), pl.ds(qidx * DQ, DQ)])

            if not last_pass:
                plsc.subcore_barrier()


@functools.partial(
    pl.kernel,
    out_type=(
        jax.ShapeDtypeStruct((N, D), jnp.float32),
        jax.ShapeDtypeStruct((4, N, DQ), jnp.float32),
        jax.ShapeDtypeStruct((4, N, DQ), jnp.float32),
    ),
    mesh=plsc.VectorSubcoreMesh(core_axis_name="c", subcore_axis_name="s"),
    compiler_params=pltpu.CompilerParams(use_tc_tiling_on_sc=False),
    scratch_types=[
        pltpu.VMEM_SHARED((N, DQ), jnp.float32),
        pltpu.VMEM_SHARED((N, L), jnp.float32),
        pltpu.VMEM((NB, C, DQ), jnp.float32),
        pltpu.VMEM((NCH, C), jnp.int32),
        pltpu.VMEM((NCH, C), jnp.int32),
        pltpu.VMEM((RPT, L), jnp.float32),
        pltpu.VMEM((NRC, DQ), jnp.float32),
        pltpu.VMEM((NRC, DQ), jnp.float32),
        pltpu.VMEM((NRC, DQ), jnp.float32),
        pltpu.VMEM((NRC, L), jnp.float32),
        pltpu.VMEM((C, L), jnp.float32),
        pltpu.SemaphoreType.DMA,
        pltpu.SemaphoreType.DMA,
    ],
)
def _lightgcn_sc(*refs):
    _lightgcn_body(*refs)


def kernel(x, edge_index):
    row2 = edge_index[0].reshape(E // C, C)
    col2 = edge_index[1].reshape(E // C, C)
    out, _, _ = _lightgcn_sc(x, row2, col2)
    return out


# speedup vs baseline: 10.8023x; 1.0306x over previous
import functools

import jax
import jax.numpy as jnp
from jax import lax
from jax.experimental import pallas as pl
from jax.experimental.pallas import tpu as pltpu
from jax.experimental.pallas import tpu_sc as plsc

N = 10000
E = 160000
D = 256
K = 3
NC = 2
NS = 16
L = 16
DQ = 64
FV = DQ // L
EPT = E // NS
C = 125
NCH = EPT // C
NB = 4
RPT = N // NS
NRC = 125
NQ = RPT // NRC


def _deg_inv_sqrt(d):
    dm = jnp.maximum(d, 1.0)
    s = dm
    for _ in range(16):
        s = 0.5 * (s + dm / s)
    return jnp.where(d > 0.0, 1.0 / s, 0.0)


def _lightgcn_body(x_hbm, row_hbm, col_hbm, out_hbm, hs_hbm, osum_hbm,
                   acc, degw,
                   gbuf, rowb, colb, sw, zbuf, zdeg, onesb,
                   sem_g, sem_s):
    nbuf = gbuf.at[1]
    osbuf = gbuf.at[2]
    c = lax.axis_index("c")
    s = lax.axis_index("s")
    r0t = s * RPT
    ech0 = s * NCH

    zero16 = jnp.zeros((L,), jnp.float32)
    one16 = jnp.ones((L,), jnp.float32)

    @pl.loop(0, NRC)
    def _(r):
        for f in range(FV):
            zbuf[r, pl.ds(f * L, L)] = zero16

    @pl.loop(0, NRC)
    def _(r):
        zdeg[r] = zero16

    @pl.loop(0, C)
    def _(r):
        onesb[r] = one16

    pltpu.sync_copy(row_hbm.at[pl.ds(ech0, NCH)], rowb)
    pltpu.sync_copy(col_hbm.at[pl.ds(ech0, NCH)], colb)

    @pl.loop(0, NQ)
    def _(q):
        pltpu.sync_copy(zbuf, acc.at[pl.ds(r0t + q * NRC, NRC)])
        pltpu.sync_copy(zdeg, degw.at[pl.ds(r0t + q * NRC, NRC)])

    plsc.subcore_barrier()

    @pl.loop(0, NCH)
    def _(j):
        pltpu.async_copy(onesb, degw.at[rowb.at[j]], sem_s, add=True)

        @pl.when(j >= 3)
        def _():
            pltpu.make_async_copy(
                onesb, degw.at[rowb.at[j - 3]], sem_s).wait()

    for t in range(NCH - 3, NCH):
        pltpu.make_async_copy(onesb, degw.at[rowb.at[t]], sem_s).wait()

    plsc.subcore_barrier()

    pltpu.sync_copy(degw.at[pl.ds(r0t, RPT)], sw)

    @pl.loop(0, RPT)
    def _(r):
        sw[r] = _deg_inv_sqrt(sw[r])

    for half in range(2):
        qidx = c * 2 + half

        @pl.loop(0, NQ)
        def _(q):
            r0g = r0t + q * NRC
            r0l = q * NRC
            pltpu.sync_copy(
                x_hbm.at[pl.ds(r0g, NRC), pl.ds(qidx * DQ, DQ)], nbuf)
            pltpu.sync_copy(nbuf, osum_hbm.at[qidx].at[pl.ds(r0g, NRC)])

            @pl.loop(0, NRC)
            def _(r):
                sv = sw[r0l + r]
                for f in range(FV):
                    df = pl.ds(f * L, L)
                    gbuf[1, r, df] = gbuf[1, r, df] * sv

            pltpu.sync_copy(nbuf, hs_hbm.at[qidx].at[pl.ds(r0g, NRC)])

    plsc.subcore_barrier()

    for k in range(1, K + 1):
        for half in range(2):
            qidx = c * 2 + half
            last_pass = (k == K) and (half == 1)

            for t in range(3):
                pltpu.async_copy(hs_hbm.at[qidx].at[colb.at[t]],
                                 gbuf.at[t], sem_g)

            @pl.loop(0, NCH)
            def _(j):
                b = lax.rem(j, NB)
                pb = lax.rem(j + NB - 1, NB)

                pltpu.make_async_copy(
                    hs_hbm.at[qidx].at[colb.at[j]], gbuf.at[b], sem_g).wait()
                pltpu.async_copy(gbuf.at[b], acc.at[rowb.at[j]], sem_s,
                                 add=True)

                @pl.when(j >= 1)
                def _():
                    pltpu.make_async_copy(
                        gbuf.at[pb], acc.at[rowb.at[j - 1]], sem_s).wait()

                @pl.when(j < NCH - 3)
                def _():
                    pltpu.async_copy(hs_hbm.at[qidx].at[colb.at[j + 3]],
                                     gbuf.at[pb], sem_g)

            lastb = (NCH - 1) % NB
            pltpu.make_async_copy(
                gbuf.at[lastb], acc.at[rowb.at[NCH - 1]], sem_s).wait()

            plsc.subcore_barrier()

            @pl.loop(0, NQ)
            def _(q):
                r0g = r0t + q * NRC
                r0l = q * NRC
                pltpu.sync_copy(acc.at[pl.ds(r0g, NRC)], nbuf)
                if not last_pass:
                    pltpu.sync_copy(zbuf, acc.at[pl.ds(r0g, NRC)])
                pltpu.sync_copy(osum_hbm.at[qidx].at[pl.ds(r0g, NRC)], osbuf)

                @pl.loop(0, NRC)
                def _(r):
                    sv = sw[r0l + r]
                    s2v = sv * sv
                    for f in range(FV):
                        df = pl.ds(f * L, L)
                        a = gbuf[1, r, df]
                        h = a * sv
                        if k < K:
                            gbuf[2, r, df] = gbuf[2, r, df] + h
                            gbuf[1, r, df] = a * s2v
                        else:
                            gbuf[1, r, df] = (gbuf[2, r, df] + h) * 0.25

                if k < K:
                    pltpu.sync_copy(osbuf,
                                    osum_hbm.at[qidx].at[pl.ds(r0g, NRC)])
                    pltpu.sync_copy(nbuf, hs_hbm.at[qidx].at[pl.ds(r0g, NRC)])
                else:
                    pltpu.sync_copy(
                        nbuf,
                        out_hbm.at[pl.ds(r0g, NRC), pl.ds(qidx * DQ, DQ)])

            if not last_pass:
                plsc.subcore_barrier()


@functools.partial(
    pl.kernel,
    out_type=(
        jax.ShapeDtypeStruct((N, D), jnp.float32),
        jax.ShapeDtypeStruct((4, N, DQ), jnp.float32),
        jax.ShapeDtypeStruct((4, N, DQ), jnp.float32),
    ),
    mesh=plsc.VectorSubcoreMesh(core_axis_name="c", subcore_axis_name="s"),
    compiler_params=pltpu.CompilerParams(use_tc_tiling_on_sc=False),
    scratch_types=[
        pltpu.VMEM_SHARED((N, DQ), jnp.float32),
        pltpu.VMEM_SHARED((N, L), jnp.float32),
        pltpu.VMEM((NB, C, DQ), jnp.float32),
        pltpu.VMEM((NCH, C), jnp.int32),
        pltpu.VMEM((NCH, C), jnp.int32),
        pltpu.VMEM((RPT, L), jnp.float32),
        pltpu.VMEM((NRC, DQ), jnp.float32),
        pltpu.VMEM((NRC, L), jnp.float32),
        pltpu.VMEM((C, L), jnp.float32),
        pltpu.SemaphoreType.DMA,
        pltpu.SemaphoreType.DMA,
    ],
)
def _lightgcn_sc(*refs):
    _lightgcn_body(*refs)


def kernel(x, edge_index):
    row2 = edge_index[0].reshape(E // C, C)
    col2 = edge_index[1].reshape(E // C, C)
    out, _, _ = _lightgcn_sc(x, row2, col2)
    return out
